# baseline (speedup 1.0000x reference)
import functools

import jax
import jax.numpy as jnp
from jax import lax
from jax.experimental import pallas as pl
from jax.experimental.pallas import tpu as pltpu
from jax.experimental.pallas import tpu_sc as plsc


def kernel(user_idx, item_idx, user_embedding, item_embedding, user_bias, item_bias):
    B = user_idx.shape[0]
    D = user_embedding.shape[1]
    info = plsc.get_sparse_core_info()
    NC, NS, L = info.num_cores, info.num_subcores, info.num_lanes
    NW = NC * NS
    assert B % (8 * NW) == 0 and D == L
    b_per_w = B // NW

    mesh = plsc.VectorSubcoreMesh(core_axis_name="c", subcore_axis_name="s")

    @functools.partial(
        pl.kernel,
        mesh=mesh,
        out_type=jax.ShapeDtypeStruct((B,), jnp.float32),
        compiler_params=pltpu.CompilerParams(
            needs_layout_passes=False, use_tc_tiling_on_sc=False),
        scratch_types=[
            pltpu.VMEM((b_per_w,), jnp.int32),
            pltpu.VMEM((b_per_w,), jnp.int32),
            pltpu.VMEM((b_per_w, D), jnp.float32),
            pltpu.VMEM((b_per_w, D), jnp.float32),
            pltpu.VMEM((b_per_w,), jnp.float32),
            pltpu.VMEM((b_per_w,), jnp.float32),
            pltpu.VMEM((b_per_w,), jnp.float32),
            pltpu.SemaphoreType.DMA,
            pltpu.SemaphoreType.DMA,
            pltpu.SemaphoreType.DMA,
            pltpu.SemaphoreType.DMA,
        ],
    )
    def run(uidx_hbm, iidx_hbm, uemb_hbm, iemb_hbm, ubias_hbm, ibias_hbm,
            out_hbm, uidx_v, iidx_v, urows_v, irows_v, ubias_v, ibias_v,
            out_v, sem_u, sem_i, sem_ub, sem_ib):
        wid = lax.axis_index("s") * NC + lax.axis_index("c")
        base = wid * b_per_w

        pltpu.sync_copy(uidx_hbm.at[pl.ds(base, b_per_w)], uidx_v)
        pltpu.sync_copy(iidx_hbm.at[pl.ds(base, b_per_w)], iidx_v)

        cu = pltpu.async_copy(uemb_hbm.at[uidx_v], urows_v, sem_u)
        ci = pltpu.async_copy(iemb_hbm.at[iidx_v], irows_v, sem_i)
        cub = pltpu.async_copy(ubias_hbm.at[uidx_v], ubias_v, sem_ub)
        cib = pltpu.async_copy(ibias_hbm.at[iidx_v], ibias_v, sem_ib)
        cu.wait()
        ci.wait()
        cub.wait()
        cib.wait()

        lane = lax.iota(jnp.int32, L)

        def body(g, carry):
            row = lane + g * L
            acc = plsc.load_gather(ubias_v, [row])
            acc = acc + plsc.load_gather(ibias_v, [row])
            for f in range(D):
                fv = jnp.full((L,), f, jnp.int32)
                acc = acc + (plsc.load_gather(urows_v, [row, fv])
                             * plsc.load_gather(irows_v, [row, fv]))
            out_v[pl.ds(g * L, L)] = acc
            return carry

        lax.fori_loop(0, b_per_w // L, body, 0)
        pltpu.sync_copy(out_v, out_hbm.at[pl.ds(base, b_per_w)])

    return run(user_idx, item_idx, user_embedding, item_embedding,
               user_bias.reshape(-1), item_bias.reshape(-1))


# speedup vs baseline: 2.4943x; 2.4943x over previous
import functools

import jax
import jax.numpy as jnp
from jax import lax
from jax.experimental import pallas as pl
from jax.experimental.pallas import tpu as pltpu
from jax.experimental.pallas import tpu_sc as plsc


def kernel(user_idx, item_idx, user_embedding, item_embedding, user_bias, item_bias):
    B = user_idx.shape[0]
    NU, D = user_embedding.shape
    NI = item_embedding.shape[0]
    info = plsc.get_sparse_core_info()
    NC, NS, L = info.num_cores, info.num_subcores, info.num_lanes
    NW = NC * NS
    assert B % (8 * NW) == 0 and D == L
    bpw = B // NW
    ngrp = bpw // L

    MAXOFF = ((NU - 128) // 128) * 128
    TAIL0 = MAXOFF + 128
    NTAIL = NU - TAIL0

    NSLOT = bpw + 4 * L

    perm = jnp.argsort(user_idx)
    uT = user_embedding.T
    item_flat = item_embedding.reshape(-1)
    ub = user_bias.reshape(-1)
    ib = item_bias.reshape(-1)
    tail_flat = user_embedding[TAIL0:].reshape(-1) if NTAIL else jnp.zeros(
        (16 * D,), jnp.float32)
    ntail_rows = max(NTAIL, 1)

    mesh = plsc.VectorSubcoreMesh(core_axis_name="c", subcore_axis_name="s")

    @functools.partial(
        pl.kernel,
        mesh=mesh,
        out_type=jax.ShapeDtypeStruct((B,), jnp.float32),
        compiler_params=pltpu.CompilerParams(
            needs_layout_passes=False, use_tc_tiling_on_sc=True),
        scratch_types=[
            pltpu.VMEM((bpw,), jnp.int32),
            pltpu.VMEM((bpw,), jnp.int32),
            pltpu.VMEM((bpw,), jnp.int32),
            pltpu.VMEM((bpw,), jnp.int32),
            pltpu.VMEM((bpw,), jnp.int32),
            pltpu.VMEM((NSLOT,), jnp.int32),
            pltpu.VMEM((NSLOT + L,), jnp.int32),
            pltpu.VMEM((2 * L, D, 128), jnp.float32),
            pltpu.VMEM((D, bpw), jnp.float32),
            pltpu.VMEM((D * bpw,), jnp.int32),
            pltpu.VMEM((D * bpw,), jnp.float32),
            pltpu.VMEM((max(NTAIL, 1) * 16,), jnp.float32),
            pltpu.VMEM((bpw,), jnp.float32),
            pltpu.VMEM((bpw,), jnp.float32),
            pltpu.VMEM((bpw,), jnp.float32),
            pltpu.SemaphoreType.DMA,
            pltpu.SemaphoreType.DMA,
        ],
    )
    def run(uidx_h, iidx_h, perm_h, uT_h, itf_h, ub_h, ib_h, tail_h,
            out_h, perm_v, us_v, is_v, cs_v, col_v, cos_v, rs_v, ring_v,
            ucolsT_v, isf_v, ir_v, tail_v, ub_v, ib_v, res_v,
            sem_ring, sem_misc):
        wid = lax.axis_index("s") * NC + lax.axis_index("c")
        base = wid * bpw
        lane = lax.iota(jnp.int32, L)

        pltpu.sync_copy(perm_h.at[pl.ds(base, bpw)], perm_v)
        cu = pltpu.async_copy(uidx_h.at[perm_v], us_v, sem_misc)
        ci = pltpu.async_copy(iidx_h.at[perm_v], is_v, sem_misc)
        ct = pltpu.async_copy(tail_h, tail_v, sem_misc)
        cu.wait()
        ci.wait()
        ct.wait()

        cub = pltpu.async_copy(ub_h.at[us_v], ub_v, sem_misc)
        cib = pltpu.async_copy(ib_h.at[is_v], ib_v, sem_misc)

        def p0a(g, carry):
            j0 = g * L
            u = us_v[pl.ds(j0, L)]
            c = u >> 7
            off = jnp.minimum(c * 128, MAXOFF)
            cs_v[pl.ds(j0, L)] = c
            col_v[pl.ds(j0, L)] = u - off
            iv = is_v[pl.ds(j0, L)] * D
            for f in range(D):
                isf_v[pl.ds(f * bpw + j0, L)] = iv + f
            return carry
        lax.fori_loop(0, ngrp, p0a, 0)

        item_copies = [
            pltpu.async_copy(itf_h.at[isf_v.at[pl.ds(f * bpw, bpw)]],
                             ir_v.at[pl.ds(f * bpw, bpw)], sem_misc)
            for f in range(D)
        ]

        def init_md(g, carry):
            j0 = g * L
            cos_v[pl.ds(j0, L)] = jnp.zeros((L,), jnp.int32)
            return carry
        lax.fori_loop(0, NSLOT // L, init_md, 0)

        def init_rs(g, carry):
            j0 = g * L
            rs_v[pl.ds(j0, L)] = jnp.full((L,), bpw, jnp.int32)
            return carry
        lax.fori_loop(0, (NSLOT + L) // L, init_rs, 0)

        def p0b(g, nslots):
            j0 = g * L
            jj = lane + j0
            cur = cs_v[pl.ds(j0, L)]
            prev = plsc.load_gather(cs_v, [jnp.maximum(jj - 1, 0)])
            nf = jnp.logical_or(cur != prev, jj == 0)
            nfi = nf.astype(jnp.int32)
            s = plsc.cumsum(nfi) - 1 + nslots
            plsc.store_scatter(cos_v, [s], cur, mask=nf)
            plsc.store_scatter(rs_v, [s], jj, mask=nf)
            return nslots + lax.reduce_sum_p.bind(nfi, axes=(0,))
        nslots = lax.fori_loop(0, ngrp, p0b, 0)
        ngroups = (nslots + L - 1) // L

        def fire_group(g):
            cvec = cos_v[pl.ds(g * L, L)]
            buf = (g % 2) * L
            for kk in range(L):
                c = cvec[kk]
                off = pl.multiple_of(jnp.minimum(c * 128, MAXOFF), 128)
                pltpu.async_copy(uT_h.at[:, pl.ds(off, 128)],
                                 ring_v.at[buf + kk], sem_ring)

        def wait_group():
            for kk in range(L):
                pltpu.make_async_copy(uT_h.at[:, pl.ds(0, 128)],
                                      ring_v.at[kk], sem_ring).wait()

        fire_group(0)

        def p1(g, carry):
            fire_group(g + 1)
            wait_group()
            buf = (g % 2) * L
            rsv = rs_v[pl.ds(g * L, L)]
            rsv1 = plsc.load_gather(rs_v, [lane + g * L + 1])
            for kk in range(L):
                j0 = rsv[kk]
                j1 = rsv1[kk]
                slotb = jnp.full((L,), buf + kk, jnp.int32)

                def pair(j, c2):
                    jb = jnp.full((L,), j, jnp.int32)
                    colb = plsc.load_gather(col_v, [jb])
                    vec = plsc.load_gather(ring_v, [slotb, lane, colb])
                    cb = plsc.load_gather(cs_v, [jb])
                    tidx = jnp.clip(colb - 128, 0, ntail_rows - 1) * D + lane
                    tvec = plsc.load_gather(tail_v, [tidx])
                    vec = jnp.where(cb >= TAIL0 // 128, tvec, vec)
                    plsc.store_scatter(ucolsT_v, [lane, jb], vec)
                    return c2
                lax.fori_loop(j0, j1, pair, 0)
            return carry
        lax.fori_loop(0, ngroups, p1, 0)
        wait_group()

        cub.wait()
        cib.wait()
        for c in item_copies:
            c.wait()

        def p2(g, carry):
            j0 = g * L
            row = lane + j0
            acc = ub_v[pl.ds(j0, L)] + ib_v[pl.ds(j0, L)]
            for f in range(D):
                uf = plsc.load_gather(ucolsT_v, [jnp.full((L,), f, jnp.int32), row])
                itf = plsc.load_gather(ir_v, [row + f * bpw])
                acc = acc + uf * itf
            res_v[pl.ds(j0, L)] = acc
            return carry
        lax.fori_loop(0, ngrp, p2, 0)

        pltpu.async_copy(res_v, out_h.at[perm_v], sem_misc).wait()

    return run(user_idx, item_idx, perm, uT, item_flat, ub, ib, tail_flat)
